# Initial kernel scaffold; baseline (speedup 1.0000x reference)
import jax
import jax.numpy as jnp
from jax.experimental import pallas as pl


def kernel(pred_cls, gt_cls):
    raise NotImplementedError("write your pallas kernel here")


# speedup vs baseline: 7.6768x; 7.6768x over previous
import jax
import jax.numpy as jnp
from jax.experimental import pallas as pl
from jax.experimental.pallas import tpu as pltpu

_N = 262144
_ROWS = _N // 128


def _body(x0_ref, x1_ref, y_ref, o_ref):
    x0 = x0_ref[...]
    x1 = x1_ref[...]
    y = y_ref[...]
    d = x1 - x0
    t = jnp.clip(y, 0, 1).astype(jnp.float32)
    z = (1.0 - 2.0 * t) * d
    nll = jnp.maximum(z, 0.0) + jnp.log1p(jnp.exp(-jnp.abs(z)))
    m = (y != -1).astype(jnp.float32)
    s = jnp.sum(nll * m)
    c = jnp.sum(m)
    o_ref[0, 0] = jnp.clip(s / jnp.maximum(c, 1.0), 0.0, 10.0)


def kernel(pred_cls, gt_cls):
    x = pred_cls.reshape(_N, 2)
    x0 = x[:, 0].reshape(_ROWS, 128)
    x1 = x[:, 1].reshape(_ROWS, 128)
    y = gt_cls.reshape(_ROWS, 128)
    out = pl.pallas_call(
        _body,
        out_shape=jax.ShapeDtypeStruct((1, 1), jnp.float32),
        out_specs=pl.BlockSpec(memory_space=pltpu.SMEM),
    )(x0, x1, y)
    return out[0, 0]
